# baseline (speedup 1.0000x reference)
import jax
import jax.numpy as jnp
from jax.experimental import pallas as pl
from jax.experimental.pallas import tpu as pltpu

B, T, V = 16, 512, 96
BLANK = V - 1
NEG = -1e30
S = 2 * T + 1
SP = 1152
TCHUNK = 128
GRID = 2
BL = B // GRID

_HI = jax.lax.Precision.HIGHEST


def _fiota(shape, dim):
    return jax.lax.broadcasted_iota(jnp.int32, shape, dim).astype(jnp.float32)


def _dot(a, b, dims):
    return jax.lax.dot_general(a, b, (dims, ((), ())), precision=_HI)


def _ctc_kernel(x_ref, labels_ref, len_ref, prob_ref,
                lp_ref, oh_ref, oh2_ref, e_ref):
    vio2 = _fiota((T, V), 1)
    path_cols = []
    for b in range(BL):
        xb = x_ref[b]
        mb = jnp.max(xb, axis=1, keepdims=True)
        exb = jnp.exp(xb - mb)
        lseb = jnp.log(jnp.sum(exb, axis=1, keepdims=True)) + mb
        lp_ref[b] = xb - lseb
        cand = jnp.where(xb >= mb, vio2, float(V))
        path_cols.append(jnp.min(cand, axis=1, keepdims=True))
    paths_t = jnp.concatenate(path_cols, axis=1)

    eio_a = _fiota((T, T), 0)
    eio_b = _fiota((T, T), 1)
    eye_t = jnp.where(eio_a == eio_b, 1.0, 0.0)
    paths = _dot(paths_t, eye_t, ((0,), (0,)))

    prev = jnp.concatenate(
        [jnp.full((BL, 1), -1.0, jnp.float32), paths[:, :-1]], axis=1)
    keep = (paths != prev) & (paths != float(BLANK))
    keep_f = jnp.where(keep, 1.0, 0.0)
    len_f = jnp.sum(keep_f, axis=1, keepdims=True)

    upper = jnp.where(eio_a <= eio_b, 1.0, 0.0)
    pos = _dot(keep_f, upper, ((1,), (0,))) - 1.0
    qos = jnp.where(keep, pos, -5.0)

    cols = []
    for b in range(BL):
        oh2_ref[...] = jnp.where(qos[b:b + 1, :] == eio_a, 1.0, 0.0)
        col = jnp.sum(oh2_ref[...] * paths[b:b + 1, :], axis=1, keepdims=True)
        cols.append(col)
    labels_t = jnp.concatenate(cols, axis=1)

    labels_f = _dot(labels_t, eye_t, ((0,), (0,)))
    labels_ref[...] = (labels_f + 0.5).astype(jnp.int32)
    len_ref[...] = (len_f + 0.5).astype(jnp.int32)

    jio_p = _fiota((T, SP), 0)
    sio_p = _fiota((T, SP), 1)
    interleave = jnp.where(sio_p == 2.0 * jio_p + 1.0, 1.0, 0.0)
    ext_odd = _dot(labels_t, interleave, ((0,), (0,)))
    s_io = _fiota((BL, SP), 1)
    is_odd = (s_io - 2.0 * jnp.floor(s_io * 0.5)) == 1.0
    ext = jnp.where(is_odd, ext_odd, float(BLANK))
    ext = jnp.where(s_io >= float(S), -2.0, ext)

    ext_m2 = jnp.concatenate(
        [jnp.full((BL, 2), -1.0, jnp.float32), ext[:, :-2]], axis=1)
    can_skip = ((s_io >= 2.0) & (ext != float(BLANK)) & (ext != ext_m2)
                & (s_io < float(S)))

    vio_sp = _fiota((V, SP), 0)
    for b in range(BL):
        oh_ref[b] = jnp.where(ext[b:b + 1, :] == vio_sp, 1.0, 0.0)

    skip01 = jnp.where(can_skip, 1.0, 0.0)
    validm = jnp.where(s_io < float(S), 1.0, 0.0)
    zcol1 = jnp.zeros((BL, 1), jnp.float32)
    zcol2 = jnp.zeros((BL, 2), jnp.float32)
    RGRP = 4

    def group(g, carry):
        a, sc = carry
        for k in range(RGRP):
            e = e_ref[g * RGRP + k]
            s1 = jnp.concatenate([zcol1, a[:, :-1]], axis=1)
            s2 = jnp.concatenate([zcol2, a[:, :-2]], axis=1)
            a = (a + s1 + skip01 * s2) * e
        m = jnp.maximum(jnp.max(a, axis=1, keepdims=True), 1e-30)
        return a * (1.0 / m), sc + jnp.log(m)

    alpha = jnp.where(s_io == 0.0, 1.0, 0.0)
    sc = jnp.zeros((BL, 1), jnp.float32)
    for c in range(T // TCHUNK):
        for b in range(BL):
            eb = _dot(lp_ref[b, c * TCHUNK:(c + 1) * TCHUNK, :], oh_ref[b],
                      ((1,), (0,)))
            e_ref[:, b, :] = jnp.exp(eb) * validm[b:b + 1, :]
        alpha, sc = jax.lax.fori_loop(0, TCHUNK // RGRP, group, (alpha, sc))

    end1 = 2.0 * len_f
    end2 = jnp.maximum(2.0 * len_f - 1.0, 0.0)
    a1 = jnp.sum(jnp.where(s_io == end1, alpha, 0.0), axis=1, keepdims=True)
    a2 = jnp.sum(jnp.where(s_io == end2, alpha, 0.0), axis=1, keepdims=True)
    a2 = jnp.where(len_f > 0.0, a2, 0.0)
    tot = a1 + a2
    prob_ref[...] = jnp.where(tot > 0.0, jnp.exp(jnp.log(tot) + sc), 0.0)


@jax.jit
def kernel(inputs):
    labels, lengths, prob = pl.pallas_call(
        _ctc_kernel,
        grid=(GRID,),
        in_specs=[pl.BlockSpec((BL, T, V), lambda i: (i, 0, 0))],
        out_specs=[
            pl.BlockSpec((BL, T), lambda i: (i, 0)),
            pl.BlockSpec((BL, 1), lambda i: (i, 0)),
            pl.BlockSpec((BL, 1), lambda i: (i, 0)),
        ],
        out_shape=[
            jax.ShapeDtypeStruct((B, T), jnp.int32),
            jax.ShapeDtypeStruct((B, 1), jnp.int32),
            jax.ShapeDtypeStruct((B, 1), jnp.float32),
        ],
        scratch_shapes=[
            pltpu.VMEM((BL, T, V), jnp.float32),
            pltpu.VMEM((BL, V, SP), jnp.float32),
            pltpu.VMEM((T, T), jnp.float32),
            pltpu.VMEM((TCHUNK, BL, SP), jnp.float32),
        ],
        compiler_params=pltpu.CompilerParams(
            dimension_semantics=("parallel",),
            vmem_limit_bytes=60 * 1024 * 1024),
    )(inputs)
    return labels, lengths.reshape(B), prob.reshape(B)


# speedup vs baseline: 1.4346x; 1.4346x over previous
import jax
import jax.numpy as jnp
from jax.experimental import pallas as pl
from jax.experimental.pallas import tpu as pltpu

B, T, V = 16, 512, 96
BLANK = V - 1
NEG = -1e30
S = 2 * T + 1
SP = 1152
TCHUNK = 128

_HI = jax.lax.Precision.HIGHEST


def _fiota(shape, dim):
    return jax.lax.broadcasted_iota(jnp.int32, shape, dim).astype(jnp.float32)


def _dot(a, b, dims):
    return jax.lax.dot_general(a, b, (dims, ((), ())), precision=_HI)


def _ctc_kernel(x_ref, labels_ref, len_ref, prob_ref,
                lp_ref, oh_ref, oh2_ref, e_ref):
    vio2 = _fiota((T, V), 1)
    path_cols = []
    for b in range(B):
        xb = x_ref[b]
        mb = jnp.max(xb, axis=1, keepdims=True)
        exb = jnp.exp(xb - mb)
        lseb = jnp.log(jnp.sum(exb, axis=1, keepdims=True)) + mb
        lp_ref[b] = xb - lseb
        cand = jnp.where(xb >= mb, vio2, float(V))
        path_cols.append(jnp.min(cand, axis=1, keepdims=True))
    paths_t = jnp.concatenate(path_cols, axis=1)

    eio_a = _fiota((T, T), 0)
    eio_b = _fiota((T, T), 1)
    eye_t = jnp.where(eio_a == eio_b, 1.0, 0.0)
    paths = _dot(paths_t, eye_t, ((0,), (0,)))

    prev = jnp.concatenate(
        [jnp.full((B, 1), -1.0, jnp.float32), paths[:, :-1]], axis=1)
    keep = (paths != prev) & (paths != float(BLANK))
    keep_f = jnp.where(keep, 1.0, 0.0)
    len_f = jnp.sum(keep_f, axis=1, keepdims=True)

    upper = jnp.where(eio_a <= eio_b, 1.0, 0.0)
    pos = _dot(keep_f, upper, ((1,), (0,))) - 1.0
    qos = jnp.where(keep, pos, -5.0)

    cols = []
    for b in range(B):
        oh2_ref[...] = jnp.where(qos[b:b + 1, :] == eio_a, 1.0, 0.0)
        col = jnp.sum(oh2_ref[...] * paths[b:b + 1, :], axis=1, keepdims=True)
        cols.append(col)
    labels_t = jnp.concatenate(cols, axis=1)

    labels_f = _dot(labels_t, eye_t, ((0,), (0,)))
    labels_ref[...] = (labels_f + 0.5).astype(jnp.int32)
    len_ref[...] = (len_f + 0.5).astype(jnp.int32)

    jio_p = _fiota((T, SP), 0)
    sio_p = _fiota((T, SP), 1)
    interleave = jnp.where(sio_p == 2.0 * jio_p + 1.0, 1.0, 0.0)
    ext_odd = _dot(labels_t, interleave, ((0,), (0,)))
    s_io = _fiota((B, SP), 1)
    is_odd = (s_io - 2.0 * jnp.floor(s_io * 0.5)) == 1.0
    ext = jnp.where(is_odd, ext_odd, float(BLANK))
    ext = jnp.where(s_io >= float(S), -2.0, ext)

    ext_m2 = jnp.concatenate(
        [jnp.full((B, 2), -1.0, jnp.float32), ext[:, :-2]], axis=1)
    can_skip = ((s_io >= 2.0) & (ext != float(BLANK)) & (ext != ext_m2)
                & (s_io < float(S)))

    vio_sp = _fiota((V, SP), 0)
    for b in range(B):
        oh_ref[b] = jnp.where(ext[b:b + 1, :] == vio_sp, 1.0, 0.0)

    skip01 = jnp.where(can_skip, 1.0, 0.0)
    validm = jnp.where(s_io < float(S), 1.0, 0.0)
    zcol1 = jnp.zeros((B, 1), jnp.float32)
    zcol2 = jnp.zeros((B, 2), jnp.float32)

    RGRP = 4

    def group(g, carry):
        a, sc = carry
        for k in range(RGRP):
            e = e_ref[g * RGRP + k]
            s1 = jnp.concatenate([zcol1, a[:, :-1]], axis=1)
            s2 = jnp.concatenate([zcol2, a[:, :-2]], axis=1)
            a = (a + s1 + skip01 * s2) * e
        m = jnp.maximum(jnp.max(a, axis=1, keepdims=True), 1e-30)
        return a * (1.0 / m), sc + jnp.log(m)

    alpha = jnp.where(s_io == 0.0, 1.0, 0.0)
    sc = jnp.zeros((B, 1), jnp.float32)
    for c in range(T // TCHUNK):
        for b in range(B):
            eb = _dot(lp_ref[b, c * TCHUNK:(c + 1) * TCHUNK, :], oh_ref[b],
                      ((1,), (0,)))
            e_ref[:, b, :] = jnp.exp(eb) * validm[b:b + 1, :]
        alpha, sc = jax.lax.fori_loop(0, TCHUNK // RGRP, group, (alpha, sc))

    end1 = 2.0 * len_f
    end2 = jnp.maximum(2.0 * len_f - 1.0, 0.0)
    a1 = jnp.sum(jnp.where(s_io == end1, alpha, 0.0), axis=1, keepdims=True)
    a2 = jnp.sum(jnp.where(s_io == end2, alpha, 0.0), axis=1, keepdims=True)
    a2 = jnp.where(len_f > 0.0, a2, 0.0)
    tot = a1 + a2
    prob_ref[...] = jnp.where(tot > 0.0, jnp.exp(jnp.log(tot) + sc), 0.0)


@jax.jit
def kernel(inputs):
    labels, lengths, prob = pl.pallas_call(
        _ctc_kernel,
        out_shape=[
            jax.ShapeDtypeStruct((B, T), jnp.int32),
            jax.ShapeDtypeStruct((B, 1), jnp.int32),
            jax.ShapeDtypeStruct((B, 1), jnp.float32),
        ],
        scratch_shapes=[
            pltpu.VMEM((B, T, V), jnp.float32),
            pltpu.VMEM((B, V, SP), jnp.float32),
            pltpu.VMEM((T, T), jnp.float32),
            pltpu.VMEM((TCHUNK, B, SP), jnp.float32),
        ],
        compiler_params=pltpu.CompilerParams(
            vmem_limit_bytes=60 * 1024 * 1024),
    )(inputs)
    return labels, lengths.reshape(B), prob.reshape(B)


# speedup vs baseline: 1.4404x; 1.0040x over previous
import jax
import jax.numpy as jnp
from jax.experimental import pallas as pl
from jax.experimental.pallas import tpu as pltpu

B, T, V = 16, 512, 96
BLANK = V - 1
NEG = -1e30
S = 2 * T + 1
SP = 1152
TCHUNK = 128

_HI = jax.lax.Precision.HIGHEST


def _fiota(shape, dim):
    return jax.lax.broadcasted_iota(jnp.int32, shape, dim).astype(jnp.float32)


def _dot(a, b, dims):
    return jax.lax.dot_general(a, b, (dims, ((), ())), precision=_HI)


def _ctc_kernel(x_ref, labels_ref, len_ref, prob_ref,
                lp_ref, oh_ref, oh2_ref, e_ref):
    vio2 = _fiota((T, V), 1)
    path_cols = []
    for b in range(B):
        xb = x_ref[b]
        mb = jnp.max(xb, axis=1, keepdims=True)
        exb = jnp.exp(xb - mb)
        lp_ref[b] = exb * (1.0 / jnp.sum(exb, axis=1, keepdims=True))
        cand = jnp.where(xb >= mb, vio2, float(V))
        path_cols.append(jnp.min(cand, axis=1, keepdims=True))
    paths_t = jnp.concatenate(path_cols, axis=1)

    eio_a = _fiota((T, T), 0)
    eio_b = _fiota((T, T), 1)
    eye_t = jnp.where(eio_a == eio_b, 1.0, 0.0)
    paths = _dot(paths_t, eye_t, ((0,), (0,)))

    prev = jnp.concatenate(
        [jnp.full((B, 1), -1.0, jnp.float32), paths[:, :-1]], axis=1)
    keep = (paths != prev) & (paths != float(BLANK))
    keep_f = jnp.where(keep, 1.0, 0.0)
    len_f = jnp.sum(keep_f, axis=1, keepdims=True)

    upper = jnp.where(eio_a <= eio_b, 1.0, 0.0)
    pos = _dot(keep_f, upper, ((1,), (0,))) - 1.0
    qos = jnp.where(keep, pos, -5.0)

    cols = []
    for b in range(B):
        oh2_ref[...] = jnp.where(qos[b:b + 1, :] == eio_a, 1.0, 0.0)
        col = jnp.sum(oh2_ref[...] * paths[b:b + 1, :], axis=1, keepdims=True)
        cols.append(col)
    labels_t = jnp.concatenate(cols, axis=1)

    labels_f = _dot(labels_t, eye_t, ((0,), (0,)))
    labels_ref[...] = (labels_f + 0.5).astype(jnp.int32)
    len_ref[...] = (len_f + 0.5).astype(jnp.int32)

    jio_p = _fiota((T, SP), 0)
    sio_p = _fiota((T, SP), 1)
    interleave = jnp.where(sio_p == 2.0 * jio_p + 1.0, 1.0, 0.0)
    ext_odd = _dot(labels_t, interleave, ((0,), (0,)))
    s_io = _fiota((B, SP), 1)
    is_odd = (s_io - 2.0 * jnp.floor(s_io * 0.5)) == 1.0
    ext = jnp.where(is_odd, ext_odd, float(BLANK))
    ext = jnp.where(s_io >= float(S), -2.0, ext)

    ext_m2 = jnp.concatenate(
        [jnp.full((B, 2), -1.0, jnp.float32), ext[:, :-2]], axis=1)
    can_skip = ((s_io >= 2.0) & (ext != float(BLANK)) & (ext != ext_m2)
                & (s_io < float(S)))

    vio_sp = _fiota((V, SP), 0)
    for b in range(B):
        oh_ref[b] = jnp.where(ext[b:b + 1, :] == vio_sp, 1.0, 0.0)

    skip01 = jnp.where(can_skip, 1.0, 0.0)
    zcol1 = jnp.zeros((B, 1), jnp.float32)
    zcol2 = jnp.zeros((B, 2), jnp.float32)

    RGRP = 4

    def group(g, carry):
        a, sc = carry
        for k in range(RGRP):
            e = e_ref[g * RGRP + k]
            s1 = jnp.concatenate([zcol1, a[:, :-1]], axis=1)
            s2 = jnp.concatenate([zcol2, a[:, :-2]], axis=1)
            a = (a + s1 + skip01 * s2) * e
        m = jnp.maximum(jnp.max(a, axis=1, keepdims=True), 1e-30)
        return a * (1.0 / m), sc + jnp.log(m)

    alpha = jnp.where(s_io == 0.0, 1.0, 0.0)
    sc = jnp.zeros((B, 1), jnp.float32)
    for c in range(T // TCHUNK):
        for b in range(B):
            eb = _dot(lp_ref[b, c * TCHUNK:(c + 1) * TCHUNK, :], oh_ref[b],
                      ((1,), (0,)))
            e_ref[:, b, :] = eb

        alpha, sc = jax.lax.fori_loop(0, TCHUNK // RGRP, group, (alpha, sc))

    end1 = 2.0 * len_f
    end2 = jnp.maximum(2.0 * len_f - 1.0, 0.0)
    a1 = jnp.sum(jnp.where(s_io == end1, alpha, 0.0), axis=1, keepdims=True)
    a2 = jnp.sum(jnp.where(s_io == end2, alpha, 0.0), axis=1, keepdims=True)
    a2 = jnp.where(len_f > 0.0, a2, 0.0)
    tot = a1 + a2
    prob_ref[...] = jnp.where(tot > 0.0, jnp.exp(jnp.log(tot) + sc), 0.0)


@jax.jit
def kernel(inputs):
    labels, lengths, prob = pl.pallas_call(
        _ctc_kernel,
        out_shape=[
            jax.ShapeDtypeStruct((B, T), jnp.int32),
            jax.ShapeDtypeStruct((B, 1), jnp.int32),
            jax.ShapeDtypeStruct((B, 1), jnp.float32),
        ],
        scratch_shapes=[
            pltpu.VMEM((B, T, V), jnp.float32),
            pltpu.VMEM((B, V, SP), jnp.float32),
            pltpu.VMEM((T, T), jnp.float32),
            pltpu.VMEM((TCHUNK, B, SP), jnp.float32),
        ],
        compiler_params=pltpu.CompilerParams(
            vmem_limit_bytes=60 * 1024 * 1024),
    )(inputs)
    return labels, lengths.reshape(B), prob.reshape(B)


# speedup vs baseline: 1.8402x; 1.2776x over previous
import jax
import jax.numpy as jnp
from jax.experimental import pallas as pl
from jax.experimental.pallas import tpu as pltpu

B, T, V = 16, 512, 96
BLANK = V - 1
S = 2 * T + 1
LO = T
LE = 640
TCHUNK = 128

_HI = jax.lax.Precision.HIGHEST


def _fiota(shape, dim):
    return jax.lax.broadcasted_iota(jnp.int32, shape, dim).astype(jnp.float32)


def _dot(a, b, dims):
    return jax.lax.dot_general(a, b, (dims, ((), ())), precision=_HI)


def _ctc_kernel(x_ref, labels_ref, len_ref, prob_ref,
                lp_ref, oh_ref, oh2_ref, e_ref, pb_ref):
    vio2 = _fiota((T, V), 1)
    path_cols = []
    for b in range(B):
        xb = x_ref[b]
        mb = jnp.max(xb, axis=1, keepdims=True)
        exb = jnp.exp(xb - mb)
        lp_ref[b] = exb * (1.0 / jnp.sum(exb, axis=1, keepdims=True))
        cand = jnp.where(xb >= mb, vio2, float(V))
        path_cols.append(jnp.min(cand, axis=1, keepdims=True))
    paths_t = jnp.concatenate(path_cols, axis=1)

    eio_a = _fiota((T, T), 0)
    eio_b = _fiota((T, T), 1)
    eye_t = jnp.where(eio_a == eio_b, 1.0, 0.0)
    paths = _dot(paths_t, eye_t, ((0,), (0,)))

    prev = jnp.concatenate(
        [jnp.full((B, 1), -1.0, jnp.float32), paths[:, :-1]], axis=1)
    keep = (paths != prev) & (paths != float(BLANK))
    keep_f = jnp.where(keep, 1.0, 0.0)
    len_f = jnp.sum(keep_f, axis=1, keepdims=True)

    upper = jnp.where(eio_a <= eio_b, 1.0, 0.0)
    pos = _dot(keep_f, upper, ((1,), (0,))) - 1.0
    qos = jnp.where(keep, pos, -5.0)

    cols = []
    for b in range(B):
        oh2_ref[...] = jnp.where(qos[b:b + 1, :] == eio_a, 1.0, 0.0)
        col = jnp.sum(oh2_ref[...] * paths[b:b + 1, :], axis=1, keepdims=True)
        cols.append(col)
    labels_t = jnp.concatenate(cols, axis=1)

    labels_f = _dot(labels_t, eye_t, ((0,), (0,)))
    labels_ref[...] = (labels_f + 0.5).astype(jnp.int32)
    len_ref[...] = (len_f + 0.5).astype(jnp.int32)

    vio_lo = _fiota((V, LO), 0)
    for b in range(B):
        oh_ref[b] = jnp.where(labels_f[b:b + 1, :] == vio_lo, 1.0, 0.0)

    for b in range(B):
        pb_ref[:, b, :] = lp_ref[b, :, BLANK:BLANK + 1]

    jio_o = _fiota((B, LO), 1)
    lab_m1 = jnp.concatenate(
        [jnp.full((B, 1), -1.0, jnp.float32), labels_f[:, :-1]], axis=1)
    skip01 = jnp.where((jio_o >= 1.0) & (labels_f != lab_m1), 1.0, 0.0)

    zcol1 = jnp.zeros((B, 1), jnp.float32)
    zpad = jnp.zeros((B, LE - LO - 1), jnp.float32)
    jio_e = _fiota((B, LE), 1)
    RGRP = 4

    ao = jnp.zeros((B, LO), jnp.float32)
    ae = jnp.where(jio_e == 0.0, 1.0, 0.0)
    sc = jnp.zeros((B, 1), jnp.float32)
    for c in range(T // TCHUNK):
        for b in range(B):
            eb = _dot(lp_ref[b, c * TCHUNK:(c + 1) * TCHUNK, :], oh_ref[b],
                      ((1,), (0,)))
            e_ref[:, b, :] = eb

        def cgroup(g, carry, c=c):
            ao, ae, sc = carry
            for k in range(RGRP):
                el = e_ref[g * RGRP + k]
                ebl = pb_ref[c * TCHUNK + g * RGRP + k]
                sh = jnp.concatenate([zcol1, ao, zpad], axis=1)
                ao = (ao + ae[:, :LO] + skip01 * sh[:, :LO]) * el
                ae = (ae + sh) * ebl
            m = jnp.maximum(jnp.max(ao, axis=1, keepdims=True),
                            jnp.max(ae, axis=1, keepdims=True))
            m = jnp.maximum(m, 1e-30)
            r = 1.0 / m
            return ao * r, ae * r, sc + jnp.log(m)

        ao, ae, sc = jax.lax.fori_loop(0, TCHUNK // RGRP, cgroup,
                                       (ao, ae, sc))

    a1 = jnp.sum(jnp.where(jio_e == len_f, ae, 0.0), axis=1, keepdims=True)
    a2 = jnp.sum(jnp.where(jio_o == len_f - 1.0, ao, 0.0),
                 axis=1, keepdims=True)
    a2 = jnp.where(len_f > 0.0, a2, 0.0)
    tot = a1 + a2
    prob_ref[...] = jnp.where(tot > 0.0, jnp.exp(jnp.log(tot) + sc), 0.0)


@jax.jit
def kernel(inputs):
    labels, lengths, prob = pl.pallas_call(
        _ctc_kernel,
        out_shape=[
            jax.ShapeDtypeStruct((B, T), jnp.int32),
            jax.ShapeDtypeStruct((B, 1), jnp.int32),
            jax.ShapeDtypeStruct((B, 1), jnp.float32),
        ],
        scratch_shapes=[
            pltpu.VMEM((B, T, V), jnp.float32),
            pltpu.VMEM((B, V, LO), jnp.float32),
            pltpu.VMEM((T, T), jnp.float32),
            pltpu.VMEM((TCHUNK, B, LO), jnp.float32),
            pltpu.VMEM((T, B, 1), jnp.float32),
        ],
        compiler_params=pltpu.CompilerParams(
            vmem_limit_bytes=60 * 1024 * 1024),
    )(inputs)
    return labels, lengths.reshape(B), prob.reshape(B)


# speedup vs baseline: 1.9846x; 1.0785x over previous
import jax
import jax.numpy as jnp
from jax.experimental import pallas as pl
from jax.experimental.pallas import tpu as pltpu

B, T, V = 16, 512, 96
BLANK = V - 1
S = 2 * T + 1
LO = T
LE = 640
TCHUNK = 128

_HI = jax.lax.Precision.HIGHEST


def _fiota(shape, dim):
    return jax.lax.broadcasted_iota(jnp.int32, shape, dim).astype(jnp.float32)


def _dot(a, b, dims):
    return jax.lax.dot_general(a, b, (dims, ((), ())), precision=_HI)


def _ctc_kernel(x_ref, labels_ref, len_ref, prob_ref,
                lp_ref, oh_ref, oh2_ref, e_ref, pb_ref):
    vio2 = _fiota((T, V), 1)
    path_cols = []
    for b in range(B):
        xb = x_ref[b]
        mb = jnp.max(xb, axis=1, keepdims=True)
        exb = jnp.exp(xb - mb)
        lp_ref[b] = exb * (1.0 / jnp.sum(exb, axis=1, keepdims=True))
        cand = jnp.where(xb >= mb, vio2, float(V))
        path_cols.append(jnp.min(cand, axis=1, keepdims=True))
    paths_t = jnp.concatenate(path_cols, axis=1)

    eio_a = _fiota((T, T), 0)
    eio_b = _fiota((T, T), 1)
    eye_t = jnp.where(eio_a == eio_b, 1.0, 0.0)
    paths = _dot(paths_t, eye_t, ((0,), (0,)))

    prev = jnp.concatenate(
        [jnp.full((B, 1), -1.0, jnp.float32), paths[:, :-1]], axis=1)
    keep = (paths != prev) & (paths != float(BLANK))
    keep_f = jnp.where(keep, 1.0, 0.0)
    len_f = jnp.sum(keep_f, axis=1, keepdims=True)

    upper = jnp.where(eio_a <= eio_b, 1.0, 0.0)
    pos = _dot(keep_f, upper, ((1,), (0,))) - 1.0
    qos = jnp.where(keep, pos, -5.0)

    cols = []
    for b in range(B):
        oh2_ref[...] = jnp.where(qos[b:b + 1, :] == eio_a, 1.0, 0.0)
        col = jnp.sum(oh2_ref[...] * paths[b:b + 1, :], axis=1, keepdims=True)
        cols.append(col)
    labels_t = jnp.concatenate(cols, axis=1)

    labels_f = _dot(labels_t, eye_t, ((0,), (0,)))
    labels_ref[...] = (labels_f + 0.5).astype(jnp.int32)
    len_ref[...] = (len_f + 0.5).astype(jnp.int32)

    vio_lo = _fiota((V, LO), 0)
    for b in range(B):
        oh_ref[b] = jnp.where(labels_f[b:b + 1, :] == vio_lo, 1.0, 0.0)

    for b in range(B):
        pb_ref[:, b, :] = lp_ref[b, :, BLANK:BLANK + 1]

    jio_o = _fiota((B, LO), 1)
    lab_m1 = jnp.concatenate(
        [jnp.full((B, 1), -1.0, jnp.float32), labels_f[:, :-1]], axis=1)
    skip01 = jnp.where((jio_o >= 1.0) & (labels_f != lab_m1), 1.0, 0.0)

    zcol1 = jnp.zeros((B, 1), jnp.float32)
    zpad = jnp.zeros((B, LE - LO - 1), jnp.float32)
    jio_e = _fiota((B, LE), 1)
    RGRP = 8

    ao = jnp.zeros((B, LO), jnp.float32)
    ae = jnp.where(jio_e == 0.0, 1.0, 0.0)
    sc = jnp.zeros((B, 1), jnp.float32)
    for c in range(T // TCHUNK):
        for b in range(B):
            eb = _dot(lp_ref[b, c * TCHUNK:(c + 1) * TCHUNK, :], oh_ref[b],
                      ((1,), (0,)))
            e_ref[:, b, :] = eb

        def cgroup(g, carry, c=c):
            ao, ae, sc = carry
            for k in range(RGRP):
                el = e_ref[g * RGRP + k]
                ebl = pb_ref[c * TCHUNK + g * RGRP + k]
                sh = jnp.concatenate([zcol1, ao, zpad], axis=1)
                ao = (ao + ae[:, :LO] + skip01 * sh[:, :LO]) * el
                ae = (ae + sh) * ebl
            m = jnp.maximum(jnp.max(ao, axis=1, keepdims=True),
                            jnp.max(ae, axis=1, keepdims=True))
            m = jnp.maximum(m, 1e-30)
            r = 1.0 / m
            return ao * r, ae * r, sc + jnp.log(m)

        ao, ae, sc = jax.lax.fori_loop(0, TCHUNK // RGRP, cgroup,
                                       (ao, ae, sc))

    a1 = jnp.sum(jnp.where(jio_e == len_f, ae, 0.0), axis=1, keepdims=True)
    a2 = jnp.sum(jnp.where(jio_o == len_f - 1.0, ao, 0.0),
                 axis=1, keepdims=True)
    a2 = jnp.where(len_f > 0.0, a2, 0.0)
    tot = a1 + a2
    prob_ref[...] = jnp.where(tot > 0.0, jnp.exp(jnp.log(tot) + sc), 0.0)


@jax.jit
def kernel(inputs):
    labels, lengths, prob = pl.pallas_call(
        _ctc_kernel,
        out_shape=[
            jax.ShapeDtypeStruct((B, T), jnp.int32),
            jax.ShapeDtypeStruct((B, 1), jnp.int32),
            jax.ShapeDtypeStruct((B, 1), jnp.float32),
        ],
        scratch_shapes=[
            pltpu.VMEM((B, T, V), jnp.float32),
            pltpu.VMEM((B, V, LO), jnp.float32),
            pltpu.VMEM((T, T), jnp.float32),
            pltpu.VMEM((TCHUNK, B, LO), jnp.float32),
            pltpu.VMEM((T, B, 1), jnp.float32),
        ],
        compiler_params=pltpu.CompilerParams(
            vmem_limit_bytes=60 * 1024 * 1024),
    )(inputs)
    return labels, lengths.reshape(B), prob.reshape(B)
